# baseline (speedup 1.0000x reference)
import jax
import jax.numpy as jnp
import numpy as np
from jax import lax
from jax.experimental import pallas as pl
from jax.experimental.pallas import tpu as pltpu
from jax.experimental.pallas import tpu_sc as plsc

N = 100000
D = 128
MAX_DEG = 10
PER_DEG = 9090
DEG0 = N - MAX_DEG * PER_DEG
STARTS = [0, DEG0] + [DEG0 + PER_DEG * k for k in range(1, MAX_DEG)]
SIZES = [DEG0] + [PER_DEG] * MAX_DEG

NW = 32
LANES = 16

CHUNK_ROWS = [64, 64, 64, 64, 64, 64, 48, 40, 40, 32, 32]
SUBGATHERS = [None, [64], [128], [96, 96], [128, 128], [80] * 4,
              [96] * 3, [56] * 5, [80] * 4, [96] * 3, [80] * 4]
GMAX = 320
CMAX = max(CHUNK_ROWS)
N_CHUNKS = [-(-SIZES[deg] // CHUNK_ROWS[deg]) for deg in range(MAX_DEG + 1)]
K_MAX = [-(-n // NW) for n in N_CHUNKS]
IMAX = max(K_MAX[deg] * CHUNK_ROWS[deg] * max(deg, 1)
           for deg in range(MAX_DEG + 1))
SMAX = max(K_MAX[deg] * CHUNK_ROWS[deg] for deg in range(MAX_DEG + 1))

ABASE = [0]
for _d in range(1, MAX_DEG + 1):
    ABASE.append(ABASE[-1] + N_CHUNKS[_d] * CHUNK_ROWS[_d] * _d)
OBASE = [0]
for _d in range(0, MAX_DEG + 1):
    OBASE.append(OBASE[-1] + N_CHUNKS[_d] * CHUNK_ROWS[_d])


def _oidx_const():
    parts = []
    for deg in range(0, MAX_DEG + 1):
        C = CHUNK_ROWS[deg]
        n_rows = SIZES[deg]
        np_rows = N_CHUNKS[deg] * C
        k = np.arange(np_rows, dtype=np.int32) // C
        r0 = np.minimum(k * C, n_rows - C)
        parts.append(STARTS[deg] + r0
                     + np.arange(np_rows, dtype=np.int32) % C)
    return np.concatenate(parts)


_OIDX = _oidx_const()


class _Deg:
    def __init__(self, deg, atoms, adj_flat, oidx_hbm, out, span, sets):
        self.deg = deg
        self.C = CHUNK_ROWS[deg]
        self.atoms, self.adj, self.oidx_hbm, self.out = (
            atoms, adj_flat, oidx_hbm, out)
        self.idx_v, self.sidx_v = span
        self.sets = sets

    def prefetch(self, lo):
        C, d, K = self.C, self.deg, K_MAX[self.deg]
        pltpu.sync_copy(self.oidx_hbm.at[pl.ds(OBASE[d] + lo * C, K * C)],
                        self.sidx_v.at[pl.ds(0, K * C)])
        if d > 0:
            pltpu.sync_copy(
                self.adj.at[pl.ds(ABASE[d - 1] + lo * C * d, K * C * d)],
                self.idx_v.at[pl.ds(0, K * C * d)])

    def _gathers(self, j, set_i):
        C, d = self.C, self.deg
        self_v, out_v, g_v, sem_g, sem_s = self.sets[set_i]
        copies = [(self.atoms.at[self.sidx_v.at[pl.ds(j * C, C)]],
                   self_v.at[pl.ds(0, C)], sem_g)]
        if d > 0:
            off = 0
            for g in SUBGATHERS[d]:
                copies.append(
                    (self.atoms.at[self.idx_v.at[pl.ds(j * C * d + off, g)]],
                     g_v.at[pl.ds(off, g)], sem_g))
                off += g
        return copies

    def fire(self, j, set_i):
        for src, dst, sem in self._gathers(j, set_i):
            pltpu.async_copy(src, dst, sem)

    def process(self, j, set_i, first):
        C, d = self.C, self.deg
        self_v, out_v, g_v, sem_g, sem_s = self.sets[set_i]
        for src, dst, sem in self._gathers(j, set_i):
            pltpu.make_async_copy(src, dst, sem).wait()

        @pl.when(jnp.logical_not(first))
        def _():
            self.wait_scatter(set_i)

        def row_body(i, _):
            for jj in range(D // LANES):
                sl = pl.ds(jj * LANES, LANES)
                acc = self_v[i, sl]
                for nb in range(d):
                    acc = jnp.maximum(acc, g_v[i * d + nb, sl])
                out_v[i, sl] = acc
            return 0

        lax.fori_loop(0, C, row_body, 0)
        pltpu.async_copy(out_v.at[pl.ds(0, C)],
                         self.out.at[self.sidx_v.at[pl.ds(j * C, C)]], sem_s)

    def wait_scatter(self, set_i):
        C = self.C
        self_v, out_v, g_v, sem_g, sem_s = self.sets[set_i]
        pltpu.make_async_copy(out_v.at[pl.ds(0, C)],
                              self.out.at[self.sidx_v.at[pl.ds(0, C)]],
                              sem_s).wait()


def _body(atoms, adj_mega, oidx_mega, out,
          idx_v, sidx_v, self_a, out_a, g_a, self_b, out_b, g_b,
          sem_ga, sem_sa, sem_gb, sem_sb):
    span = (idx_v, sidx_v)
    sets = [(self_a, out_a, g_a, sem_ga, sem_sa),
            (self_b, out_b, g_b, sem_gb, sem_sb)]
    wid = lax.axis_index("s") * 2 + lax.axis_index("c")

    for deg in range(0, MAX_DEG + 1):
        n = N_CHUNKS[deg]
        m_max = -(-K_MAX[deg] // 2)
        dd = _Deg(deg, atoms, adj_mega, oidx_mega, out, span, sets)
        lo = (wid * n) >> 5
        kw = (((wid + 1) * n) >> 5) - lo

        dd.prefetch(lo)

        @pl.when(kw > 0)
        def _(dd=dd):
            dd.fire(0, 0)

        def pipe_body(m, _, dd=dd, kw=kw):
            j0 = 2 * m
            j1 = j0 + 1
            j2 = j0 + 2

            @pl.when(j1 < kw)
            def _():
                dd.fire(j1, 1)

            @pl.when(j0 < kw)
            def _():
                dd.process(j0, 0, m < 1)

            @pl.when(j2 < kw)
            def _():
                dd.fire(j2, 0)

            @pl.when(j1 < kw)
            def _():
                dd.process(j1, 1, m < 1)

            return 0

        lax.fori_loop(0, m_max, pipe_body, 0)

        for j in range(2 * m_max):

            @pl.when(jnp.logical_and(j < kw, j + 2 >= kw))
            def _(dd=dd, j=j):
                dd.wait_scatter(j % 2)


def kernel(atom_features, deg_slice, membership, n_samples,
           deg_adj_1, deg_adj_2, deg_adj_3, deg_adj_4, deg_adj_5,
           deg_adj_6, deg_adj_7, deg_adj_8, deg_adj_9, deg_adj_10):
    del deg_slice, membership, n_samples
    adjs = [deg_adj_1, deg_adj_2, deg_adj_3, deg_adj_4, deg_adj_5,
            deg_adj_6, deg_adj_7, deg_adj_8, deg_adj_9, deg_adj_10]
    pieces = []
    for d, a in enumerate(adjs, start=1):
        C = CHUNK_ROWS[d]
        n_full = PER_DEG // C
        a2 = a.reshape(PER_DEG, d)
        flat = a2.reshape(-1)
        if PER_DEG % C:
            pieces.append(flat[: n_full * C * d])
            pieces.append(a2[PER_DEG - C:].reshape(-1))
        else:
            pieces.append(flat)
    adj_mega = jnp.concatenate(pieces)
    oidx_mega = jnp.asarray(_OIDX)

    mesh = plsc.VectorSubcoreMesh(core_axis_name="c", subcore_axis_name="s")
    run = pl.kernel(
        _body, mesh=mesh,
        out_type=jax.ShapeDtypeStruct((N, D), jnp.float32),
        scratch_types=[
            pltpu.VMEM((IMAX,), jnp.int32),
            pltpu.VMEM((SMAX,), jnp.int32),
            pltpu.VMEM((CMAX, D), jnp.float32),
            pltpu.VMEM((CMAX, D), jnp.float32),
            pltpu.VMEM((GMAX, D), jnp.float32),
            pltpu.VMEM((CMAX, D), jnp.float32),
            pltpu.VMEM((CMAX, D), jnp.float32),
            pltpu.VMEM((GMAX, D), jnp.float32),
            pltpu.SemaphoreType.DMA,
            pltpu.SemaphoreType.DMA,
            pltpu.SemaphoreType.DMA,
            pltpu.SemaphoreType.DMA,
        ],
    )
    return run(atom_features, adj_mega, oidx_mega)


# speedup vs baseline: 1.0574x; 1.0574x over previous
import jax
import jax.numpy as jnp
import numpy as np
from jax import lax
from jax.experimental import pallas as pl
from jax.experimental.pallas import tpu as pltpu
from jax.experimental.pallas import tpu_sc as plsc

N = 100000
D = 128
MAX_DEG = 10
PER_DEG = 9090
DEG0 = N - MAX_DEG * PER_DEG
STARTS = [0, DEG0] + [DEG0 + PER_DEG * k for k in range(1, MAX_DEG)]
SIZES = [DEG0] + [PER_DEG] * MAX_DEG

NW = 32
LANES = 16

CHUNK_ROWS = [64, 64, 64, 64, 64, 64, 48, 40, 40, 32, 32]
SUBGATHERS = [None, [64], [128], [96, 96], [128, 128], [80] * 4,
              [96] * 3, [56] * 5, [80] * 4, [96] * 3, [80] * 4]
GMAX = 320
CMAX = max(CHUNK_ROWS)
N_CHUNKS = [-(-SIZES[deg] // CHUNK_ROWS[deg]) for deg in range(MAX_DEG + 1)]
K_MAX = [-(-n // NW) for n in N_CHUNKS]
IMAX = max(K_MAX[deg] * CHUNK_ROWS[deg] * max(deg, 1)
           for deg in range(MAX_DEG + 1))
SMAX = max(K_MAX[deg] * CHUNK_ROWS[deg] for deg in range(MAX_DEG + 1))

ABASE = [0]
for _d in range(1, MAX_DEG + 1):
    ABASE.append(ABASE[-1] + N_CHUNKS[_d] * CHUNK_ROWS[_d] * _d)
OBASE = [0]
for _d in range(0, MAX_DEG + 1):
    OBASE.append(OBASE[-1] + N_CHUNKS[_d] * CHUNK_ROWS[_d])


def _oidx_const():
    parts = []
    for deg in range(0, MAX_DEG + 1):
        C = CHUNK_ROWS[deg]
        n_rows = SIZES[deg]
        np_rows = N_CHUNKS[deg] * C
        k = np.arange(np_rows, dtype=np.int32) // C
        r0 = np.minimum(k * C, n_rows - C)
        parts.append(STARTS[deg] + r0
                     + np.arange(np_rows, dtype=np.int32) % C)
    return np.concatenate(parts)


_OIDX = _oidx_const()


class _Deg:
    def __init__(self, deg, atoms, adj_flat, oidx_hbm, out, span, sets):
        self.deg = deg
        self.C = CHUNK_ROWS[deg]
        self.atoms, self.adj, self.oidx_hbm, self.out = (
            atoms, adj_flat, oidx_hbm, out)
        self.idx_v, self.sidx_v = span
        self.sets = sets

    def prefetch(self, lo):
        C, d, K = self.C, self.deg, K_MAX[self.deg]
        pltpu.sync_copy(self.oidx_hbm.at[pl.ds(OBASE[d] + lo * C, K * C)],
                        self.sidx_v.at[pl.ds(0, K * C)])
        if d > 0:
            pltpu.sync_copy(self.adj.at[pl.ds(lo * C * d, K * C * d)],
                            self.idx_v.at[pl.ds(0, K * C * d)])

    def _gathers(self, j, set_i):
        C, d = self.C, self.deg
        self_v, out_v, g_v, sem_g, sem_s = self.sets[set_i]
        copies = [(self.atoms.at[self.sidx_v.at[pl.ds(j * C, C)]],
                   self_v.at[pl.ds(0, C)], sem_g)]
        if d > 0:
            off = 0
            for g in SUBGATHERS[d]:
                copies.append(
                    (self.atoms.at[self.idx_v.at[pl.ds(j * C * d + off, g)]],
                     g_v.at[pl.ds(off, g)], sem_g))
                off += g
        return copies

    def fire(self, j, set_i):
        for src, dst, sem in self._gathers(j, set_i):
            pltpu.async_copy(src, dst, sem)

    def process(self, j, set_i, first):
        C, d = self.C, self.deg
        self_v, out_v, g_v, sem_g, sem_s = self.sets[set_i]
        for src, dst, sem in self._gathers(j, set_i):
            pltpu.make_async_copy(src, dst, sem).wait()

        @pl.when(jnp.logical_not(first))
        def _():
            self.wait_scatter(set_i)

        def row_body(i, _):
            for jj in range(D // LANES):
                sl = pl.ds(jj * LANES, LANES)
                acc = self_v[i, sl]
                for nb in range(d):
                    acc = jnp.maximum(acc, g_v[i * d + nb, sl])
                out_v[i, sl] = acc
            return 0

        lax.fori_loop(0, C, row_body, 0)
        pltpu.async_copy(out_v.at[pl.ds(0, C)],
                         self.out.at[self.sidx_v.at[pl.ds(j * C, C)]], sem_s)

    def wait_scatter(self, set_i):
        C = self.C
        self_v, out_v, g_v, sem_g, sem_s = self.sets[set_i]
        pltpu.make_async_copy(out_v.at[pl.ds(0, C)],
                              self.out.at[self.sidx_v.at[pl.ds(0, C)]],
                              sem_s).wait()


def _body(atoms, a1, a2, a3, a4, a5, a6, a7, a8, a9, a10, oidx_mega, out,
          idx_v, sidx_v, self_a, out_a, g_a, self_b, out_b, g_b,
          sem_ga, sem_sa, sem_gb, sem_sb):
    adj_flats = [None, a1, a2, a3, a4, a5, a6, a7, a8, a9, a10]
    span = (idx_v, sidx_v)
    sets = [(self_a, out_a, g_a, sem_ga, sem_sa),
            (self_b, out_b, g_b, sem_gb, sem_sb)]
    wid = lax.axis_index("s") * 2 + lax.axis_index("c")

    for deg in range(0, MAX_DEG + 1):
        n = N_CHUNKS[deg]
        m_max = -(-K_MAX[deg] // 2)
        dd = _Deg(deg, atoms, adj_flats[deg], oidx_mega, out, span, sets)
        lo = (wid * n) >> 5
        kw = (((wid + 1) * n) >> 5) - lo

        dd.prefetch(lo)

        @pl.when(kw > 0)
        def _(dd=dd):
            dd.fire(0, 0)

        def pipe_body(m, _, dd=dd, kw=kw):
            j0 = 2 * m
            j1 = j0 + 1
            j2 = j0 + 2

            @pl.when(j1 < kw)
            def _():
                dd.fire(j1, 1)

            @pl.when(j0 < kw)
            def _():
                dd.process(j0, 0, m < 1)

            @pl.when(j2 < kw)
            def _():
                dd.fire(j2, 0)

            @pl.when(j1 < kw)
            def _():
                dd.process(j1, 1, m < 1)

            return 0

        lax.fori_loop(0, m_max, pipe_body, 0)

        for j in range(2 * m_max):

            @pl.when(jnp.logical_and(j < kw, j + 2 >= kw))
            def _(dd=dd, j=j):
                dd.wait_scatter(j % 2)


def kernel(atom_features, deg_slice, membership, n_samples,
           deg_adj_1, deg_adj_2, deg_adj_3, deg_adj_4, deg_adj_5,
           deg_adj_6, deg_adj_7, deg_adj_8, deg_adj_9, deg_adj_10):
    del deg_slice, membership, n_samples
    adjs = [deg_adj_1, deg_adj_2, deg_adj_3, deg_adj_4, deg_adj_5,
            deg_adj_6, deg_adj_7, deg_adj_8, deg_adj_9, deg_adj_10]
    flats = []
    for d, a in enumerate(adjs, start=1):
        C = CHUNK_ROWS[d]
        n_full = PER_DEG // C
        a2 = a.reshape(PER_DEG, d)
        flat = a2.reshape(-1)
        if PER_DEG % C:
            flat = jnp.concatenate(
                [flat[: n_full * C * d], a2[PER_DEG - C:].reshape(-1)])
        flats.append(flat)
    oidx_mega = jnp.asarray(_OIDX)

    mesh = plsc.VectorSubcoreMesh(core_axis_name="c", subcore_axis_name="s")
    run = pl.kernel(
        _body, mesh=mesh,
        out_type=jax.ShapeDtypeStruct((N, D), jnp.float32),
        scratch_types=[
            pltpu.VMEM((IMAX,), jnp.int32),
            pltpu.VMEM((SMAX,), jnp.int32),
            pltpu.VMEM((CMAX, D), jnp.float32),
            pltpu.VMEM((CMAX, D), jnp.float32),
            pltpu.VMEM((GMAX, D), jnp.float32),
            pltpu.VMEM((CMAX, D), jnp.float32),
            pltpu.VMEM((CMAX, D), jnp.float32),
            pltpu.VMEM((GMAX, D), jnp.float32),
            pltpu.SemaphoreType.DMA,
            pltpu.SemaphoreType.DMA,
            pltpu.SemaphoreType.DMA,
            pltpu.SemaphoreType.DMA,
        ],
    )
    return run(atom_features, *flats, oidx_mega)


# speedup vs baseline: 1.0609x; 1.0033x over previous
import jax
import jax.numpy as jnp
import numpy as np
from jax import lax
from jax.experimental import pallas as pl
from jax.experimental.pallas import tpu as pltpu
from jax.experimental.pallas import tpu_sc as plsc

N = 100000
D = 128
MAX_DEG = 10
PER_DEG = 9090
DEG0 = N - MAX_DEG * PER_DEG
STARTS = [0, DEG0] + [DEG0 + PER_DEG * k for k in range(1, MAX_DEG)]
SIZES = [DEG0] + [PER_DEG] * MAX_DEG

NW = 32
LANES = 16

CHUNK_ROWS = [64, 64, 64, 64, 64, 64, 48, 40, 40, 32, 32]
SUBGATHERS = [None, [64], [128], [96, 96], [128, 128], [80] * 4,
              [96] * 3, [56] * 5, [80] * 4, [96] * 3, [80] * 4]
GMAX = 320
CMAX = max(CHUNK_ROWS)
N_CHUNKS = [-(-SIZES[deg] // CHUNK_ROWS[deg]) for deg in range(MAX_DEG + 1)]
K_MAX = [-(-n // NW) for n in N_CHUNKS]
IMAX = max(K_MAX[deg] * CHUNK_ROWS[deg] * max(deg, 1)
           for deg in range(MAX_DEG + 1))
SMAX = max(K_MAX[deg] * CHUNK_ROWS[deg] for deg in range(MAX_DEG + 1))

ABASE = [0]
for _d in range(1, MAX_DEG + 1):
    ABASE.append(ABASE[-1] + N_CHUNKS[_d] * CHUNK_ROWS[_d] * _d)
OBASE = [0]
for _d in range(0, MAX_DEG + 1):
    OBASE.append(OBASE[-1] + N_CHUNKS[_d] * CHUNK_ROWS[_d])


def _oidx_const():
    parts = []
    for deg in range(0, MAX_DEG + 1):
        C = CHUNK_ROWS[deg]
        n_rows = SIZES[deg]
        np_rows = N_CHUNKS[deg] * C
        k = np.arange(np_rows, dtype=np.int32) // C
        r0 = np.minimum(k * C, n_rows - C)
        parts.append(STARTS[deg] + r0
                     + np.arange(np_rows, dtype=np.int32) % C)
    return np.concatenate(parts)


_OIDX = _oidx_const()


class _Deg:
    def __init__(self, deg, atoms, adj_flat, adj_tail, oidx_hbm, out, span,
                 sets):
        self.deg = deg
        self.C = CHUNK_ROWS[deg]
        self.atoms, self.adj, self.tail, self.oidx_hbm, self.out = (
            atoms, adj_flat, adj_tail, oidx_hbm, out)
        self.idx_v, self.sidx_v = span
        self.sets = sets

    def prefetch(self, lo):
        C, d, K = self.C, self.deg, K_MAX[self.deg]
        n = N_CHUNKS[self.deg]
        pltpu.sync_copy(self.oidx_hbm.at[pl.ds(OBASE[d] + lo * C, K * C)],
                        self.sidx_v.at[pl.ds(0, K * C)])
        if d > 0:
            pltpu.sync_copy(
                self.adj.at[pl.ds(lo * C * d, (K - 1) * C * d)],
                self.idx_v.at[pl.ds(0, (K - 1) * C * d)])
            cid_last = lo + K - 1

            @pl.when(cid_last < n - 1)
            def _():
                pltpu.sync_copy(
                    self.adj.at[pl.ds(cid_last * C * d, C * d)],
                    self.idx_v.at[pl.ds((K - 1) * C * d, C * d)])

            @pl.when(cid_last == n - 1)
            def _():
                pltpu.sync_copy(
                    self.tail.at[pl.ds(0, C * d)],
                    self.idx_v.at[pl.ds((K - 1) * C * d, C * d)])

    def _gathers(self, j, set_i):
        C, d = self.C, self.deg
        self_v, out_v, g_v, sem_g, sem_s = self.sets[set_i]
        copies = [(self.atoms.at[self.sidx_v.at[pl.ds(j * C, C)]],
                   self_v.at[pl.ds(0, C)], sem_g)]
        if d > 0:
            off = 0
            for g in SUBGATHERS[d]:
                copies.append(
                    (self.atoms.at[self.idx_v.at[pl.ds(j * C * d + off, g)]],
                     g_v.at[pl.ds(off, g)], sem_g))
                off += g
        return copies

    def fire(self, j, set_i):
        for src, dst, sem in self._gathers(j, set_i):
            pltpu.async_copy(src, dst, sem)

    def process(self, j, set_i, first):
        C, d = self.C, self.deg
        self_v, out_v, g_v, sem_g, sem_s = self.sets[set_i]
        for src, dst, sem in self._gathers(j, set_i):
            pltpu.make_async_copy(src, dst, sem).wait()

        @pl.when(jnp.logical_not(first))
        def _():
            self.wait_scatter(set_i)

        def row_body(i, _):
            for jj in range(D // LANES):
                sl = pl.ds(jj * LANES, LANES)
                acc = self_v[i, sl]
                for nb in range(d):
                    acc = jnp.maximum(acc, g_v[i * d + nb, sl])
                out_v[i, sl] = acc
            return 0

        lax.fori_loop(0, C, row_body, 0)
        pltpu.async_copy(out_v.at[pl.ds(0, C)],
                         self.out.at[self.sidx_v.at[pl.ds(j * C, C)]], sem_s)

    def wait_scatter(self, set_i):
        C = self.C
        self_v, out_v, g_v, sem_g, sem_s = self.sets[set_i]
        pltpu.make_async_copy(out_v.at[pl.ds(0, C)],
                              self.out.at[self.sidx_v.at[pl.ds(0, C)]],
                              sem_s).wait()


def _body(atoms, a1, a2, a3, a4, a5, a6, a7, a8, a9, a10,
          t1, t2, t3, t4, t5, t6, t7, t8, t9, t10, oidx_mega, out,
          idx_v, sidx_v, self_a, out_a, g_a, self_b, out_b, g_b,
          sem_ga, sem_sa, sem_gb, sem_sb):
    adj_flats = [None, a1, a2, a3, a4, a5, a6, a7, a8, a9, a10]
    adj_tails = [None, t1, t2, t3, t4, t5, t6, t7, t8, t9, t10]
    span = (idx_v, sidx_v)
    sets = [(self_a, out_a, g_a, sem_ga, sem_sa),
            (self_b, out_b, g_b, sem_gb, sem_sb)]
    wid = lax.axis_index("s") * 2 + lax.axis_index("c")

    for deg in range(0, MAX_DEG + 1):
        n = N_CHUNKS[deg]
        m_max = -(-K_MAX[deg] // 2)
        dd = _Deg(deg, atoms, adj_flats[deg], adj_tails[deg], oidx_mega,
                  out, span, sets)
        lo = (wid * n) >> 5
        kw = (((wid + 1) * n) >> 5) - lo

        dd.prefetch(lo)

        @pl.when(kw > 0)
        def _(dd=dd):
            dd.fire(0, 0)

        def pipe_body(m, _, dd=dd, kw=kw):
            j0 = 2 * m
            j1 = j0 + 1
            j2 = j0 + 2

            @pl.when(j1 < kw)
            def _():
                dd.fire(j1, 1)

            @pl.when(j0 < kw)
            def _():
                dd.process(j0, 0, m < 1)

            @pl.when(j2 < kw)
            def _():
                dd.fire(j2, 0)

            @pl.when(j1 < kw)
            def _():
                dd.process(j1, 1, m < 1)

            return 0

        lax.fori_loop(0, m_max, pipe_body, 0)

        for j in range(2 * m_max):

            @pl.when(jnp.logical_and(j < kw, j + 2 >= kw))
            def _(dd=dd, j=j):
                dd.wait_scatter(j % 2)


def kernel(atom_features, deg_slice, membership, n_samples,
           deg_adj_1, deg_adj_2, deg_adj_3, deg_adj_4, deg_adj_5,
           deg_adj_6, deg_adj_7, deg_adj_8, deg_adj_9, deg_adj_10):
    del deg_slice, membership, n_samples
    adjs = [deg_adj_1, deg_adj_2, deg_adj_3, deg_adj_4, deg_adj_5,
            deg_adj_6, deg_adj_7, deg_adj_8, deg_adj_9, deg_adj_10]
    flats = []
    tails = []
    for d, a in enumerate(adjs, start=1):
        C = CHUNK_ROWS[d]
        a2 = a.reshape(PER_DEG, d)
        flats.append(a2.reshape(-1))
        tails.append(a2[PER_DEG - C:].reshape(-1))
    oidx_mega = jnp.asarray(_OIDX)

    mesh = plsc.VectorSubcoreMesh(core_axis_name="c", subcore_axis_name="s")
    run = pl.kernel(
        _body, mesh=mesh,
        out_type=jax.ShapeDtypeStruct((N, D), jnp.float32),
        scratch_types=[
            pltpu.VMEM((IMAX,), jnp.int32),
            pltpu.VMEM((SMAX,), jnp.int32),
            pltpu.VMEM((CMAX, D), jnp.float32),
            pltpu.VMEM((CMAX, D), jnp.float32),
            pltpu.VMEM((GMAX, D), jnp.float32),
            pltpu.VMEM((CMAX, D), jnp.float32),
            pltpu.VMEM((CMAX, D), jnp.float32),
            pltpu.VMEM((GMAX, D), jnp.float32),
            pltpu.SemaphoreType.DMA,
            pltpu.SemaphoreType.DMA,
            pltpu.SemaphoreType.DMA,
            pltpu.SemaphoreType.DMA,
        ],
    )
    return run(atom_features, *flats, *tails, oidx_mega)


# speedup vs baseline: 1.0633x; 1.0022x over previous
import jax
import jax.numpy as jnp
import numpy as np
from jax import lax
from jax.experimental import pallas as pl
from jax.experimental.pallas import tpu as pltpu
from jax.experimental.pallas import tpu_sc as plsc

N = 100000
D = 128
MAX_DEG = 10
PER_DEG = 9090
DEG0 = N - MAX_DEG * PER_DEG
STARTS = [0, DEG0] + [DEG0 + PER_DEG * k for k in range(1, MAX_DEG)]
SIZES = [DEG0] + [PER_DEG] * MAX_DEG

NW = 32
LANES = 16

CHUNK_ROWS = [64, 64, 64, 64, 64, 64, 48, 40, 40, 32, 32]
SUBGATHERS = [None, [64], [128], [192], [256], [320],
              [288], [280], [320], [288], [320]]
GMAX = 320
CMAX = max(CHUNK_ROWS)
N_CHUNKS = [-(-SIZES[deg] // CHUNK_ROWS[deg]) for deg in range(MAX_DEG + 1)]
K_MAX = [-(-n // NW) for n in N_CHUNKS]
IMAX = max(K_MAX[deg] * CHUNK_ROWS[deg] * max(deg, 1)
           for deg in range(MAX_DEG + 1))
SMAX = max(K_MAX[deg] * CHUNK_ROWS[deg] for deg in range(MAX_DEG + 1))

ABASE = [0]
for _d in range(1, MAX_DEG + 1):
    ABASE.append(ABASE[-1] + N_CHUNKS[_d] * CHUNK_ROWS[_d] * _d)
OBASE = [0]
for _d in range(0, MAX_DEG + 1):
    OBASE.append(OBASE[-1] + N_CHUNKS[_d] * CHUNK_ROWS[_d])


def _oidx_const():
    parts = []
    for deg in range(0, MAX_DEG + 1):
        C = CHUNK_ROWS[deg]
        n_rows = SIZES[deg]
        np_rows = N_CHUNKS[deg] * C
        k = np.arange(np_rows, dtype=np.int32) // C
        r0 = np.minimum(k * C, n_rows - C)
        parts.append(STARTS[deg] + r0
                     + np.arange(np_rows, dtype=np.int32) % C)
    return np.concatenate(parts)


_OIDX = _oidx_const()


class _Deg:
    def __init__(self, deg, atoms, adj_flat, adj_tail, oidx_hbm, out, span,
                 sets):
        self.deg = deg
        self.C = CHUNK_ROWS[deg]
        self.atoms, self.adj, self.tail, self.oidx_hbm, self.out = (
            atoms, adj_flat, adj_tail, oidx_hbm, out)
        self.idx_v, self.sidx_v = span
        self.sets = sets

    def prefetch(self, lo):
        C, d, K = self.C, self.deg, K_MAX[self.deg]
        n = N_CHUNKS[self.deg]
        pltpu.sync_copy(self.oidx_hbm.at[pl.ds(OBASE[d] + lo * C, K * C)],
                        self.sidx_v.at[pl.ds(0, K * C)])
        if d > 0:
            pltpu.sync_copy(
                self.adj.at[pl.ds(lo * C * d, (K - 1) * C * d)],
                self.idx_v.at[pl.ds(0, (K - 1) * C * d)])
            cid_last = lo + K - 1

            @pl.when(cid_last < n - 1)
            def _():
                pltpu.sync_copy(
                    self.adj.at[pl.ds(cid_last * C * d, C * d)],
                    self.idx_v.at[pl.ds((K - 1) * C * d, C * d)])

            @pl.when(cid_last == n - 1)
            def _():
                pltpu.sync_copy(
                    self.tail.at[pl.ds(0, C * d)],
                    self.idx_v.at[pl.ds((K - 1) * C * d, C * d)])

    def _gathers(self, j, set_i):
        C, d = self.C, self.deg
        self_v, out_v, g_v, sem_g, sem_s = self.sets[set_i]
        copies = [(self.atoms.at[self.sidx_v.at[pl.ds(j * C, C)]],
                   self_v.at[pl.ds(0, C)], sem_g)]
        if d > 0:
            off = 0
            for g in SUBGATHERS[d]:
                copies.append(
                    (self.atoms.at[self.idx_v.at[pl.ds(j * C * d + off, g)]],
                     g_v.at[pl.ds(off, g)], sem_g))
                off += g
        return copies

    def fire(self, j, set_i):
        for src, dst, sem in self._gathers(j, set_i):
            pltpu.async_copy(src, dst, sem)

    def process(self, j, set_i, first):
        C, d = self.C, self.deg
        self_v, out_v, g_v, sem_g, sem_s = self.sets[set_i]
        for src, dst, sem in self._gathers(j, set_i):
            pltpu.make_async_copy(src, dst, sem).wait()

        @pl.when(jnp.logical_not(first))
        def _():
            self.wait_scatter(set_i)

        def row_body(i, _):
            for jj in range(D // LANES):
                sl = pl.ds(jj * LANES, LANES)
                acc = self_v[i, sl]
                for nb in range(d):
                    acc = jnp.maximum(acc, g_v[i * d + nb, sl])
                out_v[i, sl] = acc
            return 0

        lax.fori_loop(0, C, row_body, 0)
        pltpu.async_copy(out_v.at[pl.ds(0, C)],
                         self.out.at[self.sidx_v.at[pl.ds(j * C, C)]], sem_s)

    def wait_scatter(self, set_i):
        C = self.C
        self_v, out_v, g_v, sem_g, sem_s = self.sets[set_i]
        pltpu.make_async_copy(out_v.at[pl.ds(0, C)],
                              self.out.at[self.sidx_v.at[pl.ds(0, C)]],
                              sem_s).wait()


def _body(atoms, a1, a2, a3, a4, a5, a6, a7, a8, a9, a10,
          t1, t2, t3, t4, t5, t6, t7, t8, t9, t10, oidx_mega, out,
          idx_v, sidx_v, self_a, out_a, g_a, self_b, out_b, g_b,
          sem_ga, sem_sa, sem_gb, sem_sb):
    adj_flats = [None, a1, a2, a3, a4, a5, a6, a7, a8, a9, a10]
    adj_tails = [None, t1, t2, t3, t4, t5, t6, t7, t8, t9, t10]
    span = (idx_v, sidx_v)
    sets = [(self_a, out_a, g_a, sem_ga, sem_sa),
            (self_b, out_b, g_b, sem_gb, sem_sb)]
    wid = lax.axis_index("s") * 2 + lax.axis_index("c")

    for deg in range(0, MAX_DEG + 1):
        n = N_CHUNKS[deg]
        m_max = -(-K_MAX[deg] // 2)
        dd = _Deg(deg, atoms, adj_flats[deg], adj_tails[deg], oidx_mega,
                  out, span, sets)
        lo = (wid * n) >> 5
        kw = (((wid + 1) * n) >> 5) - lo

        dd.prefetch(lo)

        @pl.when(kw > 0)
        def _(dd=dd):
            dd.fire(0, 0)

        def pipe_body(m, _, dd=dd, kw=kw):
            j0 = 2 * m
            j1 = j0 + 1
            j2 = j0 + 2

            @pl.when(j1 < kw)
            def _():
                dd.fire(j1, 1)

            @pl.when(j0 < kw)
            def _():
                dd.process(j0, 0, m < 1)

            @pl.when(j2 < kw)
            def _():
                dd.fire(j2, 0)

            @pl.when(j1 < kw)
            def _():
                dd.process(j1, 1, m < 1)

            return 0

        lax.fori_loop(0, m_max, pipe_body, 0)

        for j in range(2 * m_max):

            @pl.when(jnp.logical_and(j < kw, j + 2 >= kw))
            def _(dd=dd, j=j):
                dd.wait_scatter(j % 2)


def kernel(atom_features, deg_slice, membership, n_samples,
           deg_adj_1, deg_adj_2, deg_adj_3, deg_adj_4, deg_adj_5,
           deg_adj_6, deg_adj_7, deg_adj_8, deg_adj_9, deg_adj_10):
    del deg_slice, membership, n_samples
    adjs = [deg_adj_1, deg_adj_2, deg_adj_3, deg_adj_4, deg_adj_5,
            deg_adj_6, deg_adj_7, deg_adj_8, deg_adj_9, deg_adj_10]
    flats = []
    tails = []
    for d, a in enumerate(adjs, start=1):
        C = CHUNK_ROWS[d]
        a2 = a.reshape(PER_DEG, d)
        flats.append(a2.reshape(-1))
        tails.append(a2[PER_DEG - C:].reshape(-1))
    oidx_mega = jnp.asarray(_OIDX)

    mesh = plsc.VectorSubcoreMesh(core_axis_name="c", subcore_axis_name="s")
    run = pl.kernel(
        _body, mesh=mesh,
        out_type=jax.ShapeDtypeStruct((N, D), jnp.float32),
        scratch_types=[
            pltpu.VMEM((IMAX,), jnp.int32),
            pltpu.VMEM((SMAX,), jnp.int32),
            pltpu.VMEM((CMAX, D), jnp.float32),
            pltpu.VMEM((CMAX, D), jnp.float32),
            pltpu.VMEM((GMAX, D), jnp.float32),
            pltpu.VMEM((CMAX, D), jnp.float32),
            pltpu.VMEM((CMAX, D), jnp.float32),
            pltpu.VMEM((GMAX, D), jnp.float32),
            pltpu.SemaphoreType.DMA,
            pltpu.SemaphoreType.DMA,
            pltpu.SemaphoreType.DMA,
            pltpu.SemaphoreType.DMA,
        ],
    )
    return run(atom_features, *flats, *tails, oidx_mega)
